# baseline (speedup 1.0000x reference)
import functools

import jax
import jax.numpy as jnp
from jax import lax
from jax.experimental import pallas as pl
from jax.experimental.pallas import tpu as pltpu
from jax.experimental.pallas import tpu_sc as plsc

N = 320000
C = 128
R = 10000
NC = 2
NS = 16
NW = NC * NS
RPW = N // NW
BLK = 40
NBLK = RPW // BLK
NBUF = 5
LOOKAHEAD = 3
NITER = NBLK // NBUF
ACC_R = 10240
OUT_SLICE = ACC_R // NS
LANES = 16
WPAD = 48


def _sc_segment_sum(sem, wflat, idxflat):
    mesh = plsc.VectorSubcoreMesh(core_axis_name="c", subcore_axis_name="s")

    @functools.partial(
        pl.kernel,
        mesh=mesh,
        out_type=jax.ShapeDtypeStruct((NC, ACC_R, C), jnp.float32),
        scratch_types=[
            pltpu.VMEM((NBUF, BLK, C), jnp.float32),
            pltpu.VMEM((NBUF, WPAD), jnp.float32),
            pltpu.VMEM((NBUF, BLK), jnp.int32),
            pltpu.VMEM((2, BLK, C), jnp.float32),
            pltpu.VMEM_SHARED((ACC_R, C), jnp.float32),
        ] + [pltpu.SemaphoreType.DMA] * NBUF
          + [pltpu.SemaphoreType.DMA((2,))],
    )
    def k(sem_hbm, w_hbm, idx_hbm, out_hbm, sem_buf, w_buf, idx_buf, out_buf,
          acc, *sems):
        lsem = sems[:NBUF]
        ssem_arr = sems[NBUF]
        c = lax.axis_index("c")
        s = lax.axis_index("s")
        wid = s * NC + c
        row0 = wid * RPW

        def issue_loads(jj, b):
            base = row0 + jj * BLK
            pltpu.async_copy(sem_hbm.at[pl.ds(base, BLK)], sem_buf.at[b],
                             lsem[b])
            pltpu.async_copy(idx_hbm.at[pl.ds(base, BLK)], idx_buf.at[b],
                             lsem[b])
            pltpu.async_copy(w_hbm.at[pl.ds(base, BLK)],
                             w_buf.at[b, pl.ds(0, BLK)], lsem[b])

        def wait_loads(jj, b):
            base = row0 + jj * BLK
            pltpu.make_async_copy(sem_hbm.at[pl.ds(base, BLK)],
                                  sem_buf.at[b], lsem[b]).wait()
            pltpu.make_async_copy(idx_hbm.at[pl.ds(base, BLK)],
                                  idx_buf.at[b], lsem[b]).wait()
            pltpu.make_async_copy(w_hbm.at[pl.ds(base, BLK)],
                                  w_buf.at[b, pl.ds(0, BLK)], lsem[b]).wait()

        def start_scatter(d2, b):
            pltpu.async_copy(out_buf.at[d2], acc.at[idx_buf.at[b]],
                             ssem_arr.at[d2], add=True)

        def wait_scatter(d2):
            pltpu.make_async_copy(out_buf.at[d2], acc.at[idx_buf.at[0]],
                                  ssem_arr.at[d2]).wait()

        def scale_block(d2, b):
            for g in range((BLK + LANES - 1) // LANES):
                wv = w_buf[b, pl.ds(g * LANES, LANES)]
                for l in range(min(LANES, BLK - g * LANES)):
                    r = g * LANES + l
                    wb = jnp.broadcast_to(wv[l], (LANES,))
                    for h in range(C // LANES):
                        sl = pl.ds(h * LANES, LANES)
                        out_buf[d2, r, sl] = sem_buf[b, r, sl] * wb

        zero16 = jnp.zeros((LANES,), jnp.float32)
        for i in range(BLK):
            for h in range(C // LANES):
                out_buf[0, i, pl.ds(h * LANES, LANES)] = zero16
        for t in range(OUT_SLICE // BLK):
            pltpu.async_copy(out_buf.at[0],
                             acc.at[pl.ds(s * OUT_SLICE + t * BLK, BLK)],
                             lsem[0])
        for t in range(OUT_SLICE // BLK):
            pltpu.make_async_copy(
                out_buf.at[0],
                acc.at[pl.ds(s * OUT_SLICE + t * BLK, BLK)],
                lsem[0]).wait()
        plsc.subcore_barrier()

        for b in range(LOOKAHEAD):
            issue_loads(b, b)

        def body(i, carry):
            for u in range(NBUF):
                j = NBUF * i + u
                d2 = lax.rem(j, 2)
                wait_loads(j, u)
                if u < 2:
                    @pl.when(i >= 1)
                    def _():
                        wait_scatter(d2)
                else:
                    wait_scatter(d2)
                tb = (u + LOOKAHEAD) % NBUF
                if u + LOOKAHEAD < NBUF:
                    issue_loads(j + LOOKAHEAD, tb)
                else:
                    @pl.when(i <= NITER - 2)
                    def _():
                        issue_loads(j + LOOKAHEAD, tb)
                scale_block(d2, u)
                start_scatter(d2, u)
            return carry

        lax.fori_loop(0, NITER, body, 0)

        for d2 in range(2):
            wait_scatter(d2)
        plsc.subcore_barrier()
        pltpu.sync_copy(
            acc.at[pl.ds(s * OUT_SLICE, OUT_SLICE)],
            out_hbm.at[c].at[pl.ds(s * OUT_SLICE, OUT_SLICE)],
        )

    return k(sem, wflat, idxflat)


def _tc_combine(partial):
    def body(p_ref, o_ref):
        o_ref[...] = p_ref[0] + p_ref[1]

    blk = 1000
    return pl.pallas_call(
        body,
        grid=(R // blk,),
        in_specs=[pl.BlockSpec((NC, blk, C), lambda i: (0, i, 0))],
        out_specs=pl.BlockSpec((blk, C), lambda i: (i, 0)),
        out_shape=jax.ShapeDtypeStruct((R, C), jnp.float32),
    )(partial)


def kernel(semantics, weights, ray_indices, num_rays):
    idx = jnp.minimum(ray_indices,
                      jnp.asarray(num_rays, ray_indices.dtype) - 1)
    partial = _sc_segment_sum(semantics, weights.reshape(N), idx)
    return _tc_combine(partial)


# speedup vs baseline: 3.1780x; 3.1780x over previous
import functools

import jax
import jax.numpy as jnp
from jax import lax
from jax.experimental import pallas as pl
from jax.experimental.pallas import tpu as pltpu
from jax.experimental.pallas import tpu_sc as plsc

N = 320000
C = 128
R = 10000
NC = 2
NS = 16
NW = NC * NS
RPW = N // NW
BLK = 40
NBLK = RPW // BLK
NBUF = 5
LOOKAHEAD = 3
NITER = NBLK // NBUF
ACC_R = 10240
OUT_SLICE = ACC_R // NS
LANES = 16
WPAD = 48


def _sc_segment_sum(sem, wflat, idxflat):
    mesh = plsc.VectorSubcoreMesh(core_axis_name="c", subcore_axis_name="s")

    @functools.partial(
        pl.kernel,
        mesh=mesh,
        out_type=jax.ShapeDtypeStruct((NC, ACC_R, C), jnp.float32),
        scratch_types=[
            pltpu.VMEM((NBUF, BLK, C), jnp.float32),
            pltpu.VMEM((NBUF, WPAD), jnp.float32),
            pltpu.VMEM((NBUF, BLK), jnp.int32),
            pltpu.VMEM_SHARED((ACC_R, C), jnp.float32),
        ] + [pltpu.SemaphoreType.DMA] * (2 * NBUF),
    )
    def k(sem_hbm, w_hbm, idx_hbm, out_hbm, sem_buf, w_buf, idx_buf, acc,
          *sems):
        lsem = sems[:NBUF]
        ssem = sems[NBUF:]
        c = lax.axis_index("c")
        s = lax.axis_index("s")
        wid = s * NC + c
        row0 = wid * RPW

        def issue_loads(jj, b):
            base = row0 + jj * BLK
            pltpu.async_copy(sem_hbm.at[pl.ds(base, BLK)], sem_buf.at[b],
                             lsem[b])
            pltpu.async_copy(idx_hbm.at[pl.ds(base, BLK)], idx_buf.at[b],
                             lsem[b])
            pltpu.async_copy(w_hbm.at[pl.ds(base, BLK)],
                             w_buf.at[b, pl.ds(0, BLK)], lsem[b])

        def wait_loads(jj, b):
            base = row0 + jj * BLK
            pltpu.make_async_copy(sem_hbm.at[pl.ds(base, BLK)],
                                  sem_buf.at[b], lsem[b]).wait()
            pltpu.make_async_copy(idx_hbm.at[pl.ds(base, BLK)],
                                  idx_buf.at[b], lsem[b]).wait()
            pltpu.make_async_copy(w_hbm.at[pl.ds(base, BLK)],
                                  w_buf.at[b, pl.ds(0, BLK)], lsem[b]).wait()

        def start_scatter(b):
            pltpu.async_copy(sem_buf.at[b], acc.at[idx_buf.at[b]], ssem[b],
                             add=True)

        def wait_scatter(b):
            pltpu.make_async_copy(sem_buf.at[b], acc.at[idx_buf.at[b]],
                                  ssem[b]).wait()

        def scale_block(b):
            for g in range((BLK + LANES - 1) // LANES):
                wv = w_buf[b, pl.ds(g * LANES, LANES)]
                for l in range(min(LANES, BLK - g * LANES)):
                    r = g * LANES + l
                    wb = jnp.broadcast_to(wv[l], (LANES,))
                    for h in range(C // LANES):
                        sl = pl.ds(h * LANES, LANES)
                        sem_buf[b, r, sl] = sem_buf[b, r, sl] * wb

        zero16 = jnp.zeros((LANES,), jnp.float32)
        for i in range(BLK):
            for h in range(C // LANES):
                sem_buf[0, i, pl.ds(h * LANES, LANES)] = zero16
        for t in range(OUT_SLICE // BLK):
            pltpu.async_copy(sem_buf.at[0],
                             acc.at[pl.ds(s * OUT_SLICE + t * BLK, BLK)],
                             ssem[0])
        for t in range(OUT_SLICE // BLK):
            pltpu.make_async_copy(
                sem_buf.at[0],
                acc.at[pl.ds(s * OUT_SLICE + t * BLK, BLK)],
                ssem[0]).wait()
        plsc.subcore_barrier()

        for b in range(LOOKAHEAD):
            issue_loads(b, b)

        def body(i, carry):
            for u in range(NBUF):
                j = NBUF * i + u
                wait_loads(j, u)
                tb = (u + LOOKAHEAD) % NBUF
                if u + LOOKAHEAD < NBUF:
                    @pl.when(i >= 1)
                    def _():
                        wait_scatter(tb)

                    issue_loads(j + LOOKAHEAD, tb)
                else:
                    @pl.when(i <= NITER - 2)
                    def _():
                        wait_scatter(tb)
                        issue_loads(j + LOOKAHEAD, tb)
                scale_block(u)
                start_scatter(u)
            return carry

        lax.fori_loop(0, NITER, body, 0)

        for b in range(NBUF):
            wait_scatter(b)
        plsc.subcore_barrier()
        pltpu.sync_copy(
            acc.at[pl.ds(s * OUT_SLICE, OUT_SLICE)],
            out_hbm.at[c].at[pl.ds(s * OUT_SLICE, OUT_SLICE)],
        )

    return k(sem, wflat, idxflat)


def _tc_combine(partial):
    def body(p_ref, o_ref):
        o_ref[...] = p_ref[0] + p_ref[1]

    blk = 1000
    return pl.pallas_call(
        body,
        grid=(R // blk,),
        in_specs=[pl.BlockSpec((NC, blk, C), lambda i: (0, i, 0))],
        out_specs=pl.BlockSpec((blk, C), lambda i: (i, 0)),
        out_shape=jax.ShapeDtypeStruct((R, C), jnp.float32),
    )(partial)


def kernel(semantics, weights, ray_indices, num_rays):
    idx = jnp.minimum(ray_indices,
                      jnp.asarray(num_rays, ray_indices.dtype) - 1)
    partial = _sc_segment_sum(semantics, weights.reshape(N), idx)
    return _tc_combine(partial)


# speedup vs baseline: 3.2073x; 1.0092x over previous
import functools

import jax
import jax.numpy as jnp
from jax import lax
from jax.experimental import pallas as pl
from jax.experimental.pallas import tpu as pltpu
from jax.experimental.pallas import tpu_sc as plsc

N = 320000
C = 128
R = 10000
NC = 2
NS = 16
NW = NC * NS
RPW = N // NW
BLK = 40
NBLK = RPW // BLK
NBUF = 5
LOOKAHEAD = 3
NITER = NBLK // NBUF
ACC_R = 10240
OUT_SLICE = ACC_R // NS
LANES = 16
WPAD = 48


def _sc_segment_sum(sem, wflat, idxflat):
    mesh = plsc.VectorSubcoreMesh(core_axis_name="c", subcore_axis_name="s")

    @functools.partial(
        pl.kernel,
        mesh=mesh,
        out_type=jax.ShapeDtypeStruct((NC, ACC_R, C), jnp.float32),
        scratch_types=[
            pltpu.VMEM((NBUF, BLK, C), jnp.float32),
            pltpu.VMEM((NBUF, WPAD), jnp.float32),
            pltpu.VMEM((NBUF, BLK), jnp.int32),
            pltpu.VMEM_SHARED((ACC_R, C), jnp.float32),
        ] + [pltpu.SemaphoreType.DMA] * (2 * NBUF),
    )
    def k(sem_hbm, w_hbm, idx_hbm, out_hbm, sem_buf, w_buf, idx_buf, acc,
          *sems):
        lsem = sems[:NBUF]
        ssem = sems[NBUF:]
        c = lax.axis_index("c")
        s = lax.axis_index("s")
        wid = s * NC + c
        row0 = wid * RPW

        def issue_loads(jj, b):
            base = row0 + jj * BLK
            pltpu.async_copy(sem_hbm.at[pl.ds(base, BLK)], sem_buf.at[b],
                             lsem[b])
            pltpu.async_copy(idx_hbm.at[pl.ds(base, BLK)], idx_buf.at[b],
                             lsem[b])
            pltpu.async_copy(w_hbm.at[pl.ds(base, BLK)],
                             w_buf.at[b, pl.ds(0, BLK)], lsem[b])

        def wait_loads(jj, b):
            base = row0 + jj * BLK
            pltpu.make_async_copy(sem_hbm.at[pl.ds(base, BLK)],
                                  sem_buf.at[b], lsem[b]).wait()
            pltpu.make_async_copy(idx_hbm.at[pl.ds(base, BLK)],
                                  idx_buf.at[b], lsem[b]).wait()
            pltpu.make_async_copy(w_hbm.at[pl.ds(base, BLK)],
                                  w_buf.at[b, pl.ds(0, BLK)], lsem[b]).wait()

        def start_scatter(b):
            pltpu.async_copy(sem_buf.at[b], acc.at[idx_buf.at[b]], ssem[b],
                             add=True)

        def wait_scatter(b):
            pltpu.make_async_copy(sem_buf.at[b], acc.at[idx_buf.at[b]],
                                  ssem[b]).wait()

        def scale_block(b):
            for g in range((BLK + LANES - 1) // LANES):
                wv = w_buf[b, pl.ds(g * LANES, LANES)]
                for l in range(min(LANES, BLK - g * LANES)):
                    r = g * LANES + l
                    wb = jnp.broadcast_to(wv[l], (LANES,))
                    for h in range(C // LANES):
                        sl = pl.ds(h * LANES, LANES)
                        sem_buf[b, r, sl] = sem_buf[b, r, sl] * wb

        zero16 = jnp.zeros((LANES,), jnp.float32)
        for i in range(BLK):
            for h in range(C // LANES):
                sem_buf[NBUF - 1, i, pl.ds(h * LANES, LANES)] = zero16
        for t in range(OUT_SLICE // BLK):
            pltpu.async_copy(sem_buf.at[NBUF - 1],
                             acc.at[pl.ds(s * OUT_SLICE + t * BLK, BLK)],
                             lsem[NBUF - 1])
        for b in range(LOOKAHEAD):
            issue_loads(b, b)
        for t in range(OUT_SLICE // BLK):
            pltpu.make_async_copy(
                sem_buf.at[NBUF - 1],
                acc.at[pl.ds(s * OUT_SLICE + t * BLK, BLK)],
                lsem[NBUF - 1]).wait()
        plsc.subcore_barrier()

        def body(i, carry):
            for u in range(NBUF):
                j = NBUF * i + u
                wait_loads(j, u)
                tb = (u + LOOKAHEAD) % NBUF
                if u + LOOKAHEAD < NBUF:
                    @pl.when(i >= 1)
                    def _():
                        wait_scatter(tb)

                    issue_loads(j + LOOKAHEAD, tb)
                else:
                    @pl.when(i <= NITER - 2)
                    def _():
                        wait_scatter(tb)
                        issue_loads(j + LOOKAHEAD, tb)
                scale_block(u)
                start_scatter(u)
            return carry

        lax.fori_loop(0, NITER, body, 0)

        for b in range(NBUF):
            wait_scatter(b)
        plsc.subcore_barrier()
        pltpu.sync_copy(
            acc.at[pl.ds(s * OUT_SLICE, OUT_SLICE)],
            out_hbm.at[c].at[pl.ds(s * OUT_SLICE, OUT_SLICE)],
        )

    return k(sem, wflat, idxflat)


def _tc_combine(partial):
    def body(p_ref, o_ref):
        o_ref[...] = p_ref[0] + p_ref[1]

    blk = 1000
    return pl.pallas_call(
        body,
        grid=(R // blk,),
        in_specs=[pl.BlockSpec((NC, blk, C), lambda i: (0, i, 0))],
        out_specs=pl.BlockSpec((blk, C), lambda i: (i, 0)),
        out_shape=jax.ShapeDtypeStruct((R, C), jnp.float32),
    )(partial)


def kernel(semantics, weights, ray_indices, num_rays):
    idx = jnp.minimum(ray_indices,
                      jnp.asarray(num_rays, ray_indices.dtype) - 1)
    partial = _sc_segment_sum(semantics, weights.reshape(N), idx)
    return _tc_combine(partial)
